# Initial kernel scaffold; baseline (speedup 1.0000x reference)
import jax
import jax.numpy as jnp
from jax.experimental import pallas as pl


def kernel(x, grid, grid_weight, edge_grid, edge_Gauss, basepts, base_weight, D, weights):
    raise NotImplementedError("write your pallas kernel here")


# speedup vs baseline: 5.9838x; 5.9838x over previous
import functools

import jax
import jax.numpy as jnp
from jax import lax
from jax.experimental import pallas as pl
from jax.experimental.pallas import tpu as pltpu
from jax.experimental.pallas import tpu_sc as plsc

BSZ = 4
N = 10000
PHY = 3
NP = 1024
K = 64
C = 32
KM = 16
L = 16

NCORE = 2
NSUB = 16
BPC = BSZ // NCORE
PPT = NP // NSUB
PAIRS = BPC * PPT


def _vsqrt(sv):
    iv = plsc.bitcast(sv, jnp.int32)
    iv = lax.shift_right_logical(iv, 1) + 0x1FBD1DF5
    y = plsc.bitcast(iv, jnp.float32)
    for _ in range(3):
        y = 0.5 * (y + sv / y)
    return y


def _sc_k1_body(gpl_hbm, cpl_hbm, eg_hbm, eq_hbm, xp_hbm, z_hbm,
                xhat_hbm, gauss_hbm,
                gpl_l, cpl_l, eg_l, eq_l, gidx_l, qg_l, gbuf, wg_l, xrows,
                xhat_sh, sem):
    c = lax.axis_index("c")
    s = lax.axis_index("s")
    iota = lax.iota(jnp.int32, L)

    for d in range(4):
        for j in range(BPC):
            pltpu.sync_copy(gpl_hbm.at[pl.ds((d * BSZ + 2 * c + j) * N, N)],
                            gpl_l.at[pl.ds((d * BPC + j) * N, N)])
    pltpu.sync_copy(cpl_hbm, cpl_l)

    @pl.when(s == 0)
    def _():
        pltpu.sync_copy(z_hbm, xhat_sh)
    plsc.subcore_barrier()

    def pair_body(b_l, p_l):
        p = s * PPT + p_l
        b = 2 * c + b_l
        pair_g = b * NP + p
        pltpu.sync_copy(eg_hbm.at[pl.ds(pair_g * K, K)], eg_l)
        pltpu.sync_copy(eq_hbm.at[pl.ds(pair_g * K, K)], eq_l)

        bwx = plsc.load_gather(cpl_l, [jnp.full((L,), 3 * NP, jnp.int32) + p])
        bwy = plsc.load_gather(cpl_l, [jnp.full((L,), 4 * NP, jnp.int32) + p])
        bwz = plsc.load_gather(cpl_l, [jnp.full((L,), 5 * NP, jnp.int32) + p])

        acc = jnp.zeros((L,), jnp.float32)
        for blk in range(K // L):
            ge = eg_l[pl.ds(blk * L, L)]
            qe = eq_l[pl.ds(blk * L, L)]
            geb = ge + b_l * N
            gx = plsc.load_gather(gpl_l, [geb])
            gy = plsc.load_gather(gpl_l, [geb + BPC * N])
            gz = plsc.load_gather(gpl_l, [geb + 2 * BPC * N])
            gw = plsc.load_gather(gpl_l, [geb + 3 * BPC * N])
            bx = plsc.load_gather(cpl_l, [qe])
            by = plsc.load_gather(cpl_l, [qe + NP])
            bz = plsc.load_gather(cpl_l, [qe + 2 * NP])
            dx = gx - bx
            dy = gy - by
            dz = gz - bz
            dw = bwx * dx * dx + bwy * dy * dy + bwz * dz * dz
            gv = jnp.exp(-dw)
            wv = gv * gw
            gbuf[pl.ds(blk * L, L)] = gv
            acc = acc + wv * wv
            wg_l[pl.ds(blk * L, L)] = wv
            gidx_l[pl.ds(blk * L, L)] = ge + b * N
            qg_l[pl.ds(blk * L, L)] = qe + b_l * NP

        tot = jnp.sum(acc, axis=0)
        nrm = _vsqrt(jnp.full((L,), 0.0, jnp.float32) + tot) + 1e-5
        winv = 1.0 / nrm
        wgn = [wg_l[pl.ds(blk * L, L)] * winv for blk in range(K // L)]

        pltpu.async_copy(xp_hbm.at[gidx_l], xrows, sem).wait()

        for blk in range(K // L):
            ridx = iota + blk * L
            wv = wgn[blk]
            for ch in range(C):
                cfull = jnp.full((L,), ch, jnp.int32)
                v = plsc.load_gather(xrows, [ridx, cfull])
                plsc.store_scatter(xrows, [ridx, cfull], v * wv)

        pltpu.sync_copy(xrows, xhat_sh.at[qg_l], add=True)
        pltpu.sync_copy(gbuf, gauss_hbm.at[pl.ds(pair_g * K, K)])
        return p_l + 1

    for b_l in range(BPC):
        lax.fori_loop(0, PPT, lambda i, _, b_l=b_l: pair_body(b_l, i), 0)

    plsc.subcore_barrier()
    rows = BPC * NP // NSUB
    pltpu.sync_copy(xhat_sh.at[pl.ds(s * rows, rows)],
                    xhat_hbm.at[pl.ds(c * BPC * NP + s * rows, rows)])


def _sc_k3_body(y_hbm, gauss_hbm, eg_hbm, eq_hbm, z_hbm,
                out_hbm,
                y_l, eg_l, eq_l, gidx_l, gbuf, rows_v,
                out_sh, sem):
    c = lax.axis_index("c")
    s = lax.axis_index("s")
    iota = lax.iota(jnp.int32, L)
    zrows = BPC * N // NSUB

    pltpu.sync_copy(y_hbm.at[pl.ds(c * BPC * NP * C, BPC * NP * C)], y_l)

    @pl.when(s == 0)
    def _():
        pltpu.sync_copy(z_hbm, out_sh)
    plsc.subcore_barrier()

    def pair_body(b_l, p_l):
        p = s * PPT + p_l
        b = 2 * c + b_l
        pair_g = b * NP + p
        pltpu.sync_copy(eg_hbm.at[pl.ds(pair_g * K, K)], eg_l)
        pltpu.sync_copy(eq_hbm.at[pl.ds(pair_g * K, K)], eq_l)
        pltpu.sync_copy(gauss_hbm.at[pl.ds(pair_g * K, K)], gbuf)

        for blk in range(K // L):
            ge = eg_l[pl.ds(blk * L, L)]
            qrow = (eq_l[pl.ds(blk * L, L)] + b_l * NP) * C
            gv = gbuf[pl.ds(blk * L, L)]
            ridx = iota + blk * L
            gidx_l[pl.ds(blk * L, L)] = ge + b_l * N
            for ch in range(C):
                cfull = jnp.full((L,), ch, jnp.int32)
                v = plsc.load_gather(y_l, [qrow + ch])
                plsc.store_scatter(rows_v, [ridx, cfull], v * gv)

        pltpu.sync_copy(rows_v, out_sh.at[gidx_l], add=True)
        return p_l + 1

    for b_l in range(BPC):
        lax.fori_loop(0, PPT, lambda i, _, b_l=b_l: pair_body(b_l, i), 0)

    plsc.subcore_barrier()
    pltpu.sync_copy(out_sh.at[pl.ds(s * zrows, zrows)],
                    out_hbm.at[pl.ds(c * BPC * N + s * zrows, zrows)])


def _tc_k2_body(xh_ref, w2_ref, dt_ref, y_ref):
    t = jnp.dot(xh_ref[...], w2_ref[...], preferred_element_type=jnp.float32)
    acc = jnp.zeros_like(y_ref)
    for j in range(KM):
        acc = acc + t[:, j * C:(j + 1) * C] * dt_ref[:, j:j + 1]
    y_ref[...] = acc


@jax.jit
def kernel(x, grid, grid_weight, edge_grid, edge_Gauss, basepts, base_weight, D, weights):
    gpl = jnp.concatenate([jnp.transpose(grid, (2, 0, 1)),
                           grid_weight[None]], axis=0).reshape(-1)
    cpl = jnp.concatenate([basepts.T, base_weight.T,
                           jnp.zeros((2, NP), jnp.float32)], axis=0)
    cpl_flat = cpl.reshape(-1)
    eg_flat = edge_grid.reshape(-1)
    eq_flat = edge_Gauss.reshape(-1)
    xp = jnp.transpose(x, (0, 2, 1)).reshape(BSZ * N, C)
    z1 = jnp.zeros((BPC * NP, C), jnp.float32)
    z3 = jnp.zeros((BPC * N, C), jnp.float32)

    mesh = plsc.VectorSubcoreMesh(core_axis_name="c", subcore_axis_name="s")
    sc_params = pltpu.CompilerParams(needs_layout_passes=False,
                                     use_tc_tiling_on_sc=False)

    k1 = pl.kernel(
        _sc_k1_body,
        compiler_params=sc_params,
        out_type=[jax.ShapeDtypeStruct((BSZ * NP, C), jnp.float32),
                  jax.ShapeDtypeStruct((BSZ * NP * K,), jnp.float32)],
        mesh=mesh,
        scratch_types=[
            pltpu.VMEM((4 * BPC * N,), jnp.float32),
            pltpu.VMEM((8 * NP,), jnp.float32),
            pltpu.VMEM((K,), jnp.int32),
            pltpu.VMEM((K,), jnp.int32),
            pltpu.VMEM((K,), jnp.int32),
            pltpu.VMEM((K,), jnp.int32),
            pltpu.VMEM((K,), jnp.float32),
            pltpu.VMEM((K,), jnp.float32),
            pltpu.VMEM((K, C), jnp.float32),
            pltpu.MemorySpace.VMEM_SHARED((BPC * NP, C), jnp.float32),
            pltpu.SemaphoreType.DMA,
        ],
    )
    x_hat, gauss = k1(gpl, cpl_flat, eg_flat, eq_flat, xp, z1)

    w2 = jnp.transpose(weights, (0, 2, 1)).reshape(C, KM * C)
    dt = D.T
    RB = 512
    y = pl.pallas_call(
        _tc_k2_body,
        grid=(BSZ * NP // RB,),
        in_specs=[
            pl.BlockSpec((RB, C), lambda i: (i, 0)),
            pl.BlockSpec((C, KM * C), lambda i: (0, 0)),
            pl.BlockSpec((RB, KM), lambda i: (i % (NP // RB), 0)),
        ],
        out_specs=pl.BlockSpec((RB, C), lambda i: (i, 0)),
        out_shape=jax.ShapeDtypeStruct((BSZ * NP, C), jnp.float32),
    )(x_hat, w2, dt)

    k3 = pl.kernel(
        _sc_k3_body,
        compiler_params=sc_params,
        out_type=jax.ShapeDtypeStruct((BSZ * N, C), jnp.float32),
        mesh=mesh,
        scratch_types=[
            pltpu.VMEM((BPC * NP * C,), jnp.float32),
            pltpu.VMEM((K,), jnp.int32),
            pltpu.VMEM((K,), jnp.int32),
            pltpu.VMEM((K,), jnp.int32),
            pltpu.VMEM((K,), jnp.float32),
            pltpu.VMEM((K, C), jnp.float32),
            pltpu.MemorySpace.VMEM_SHARED((BPC * N, C), jnp.float32),
            pltpu.SemaphoreType.DMA,
        ],
    )
    out = k3(y.reshape(-1), gauss, eg_flat, eq_flat, z3)
    return jnp.transpose(out.reshape(BSZ, N, C), (0, 2, 1))


# speedup vs baseline: 8.4174x; 1.4067x over previous
import functools

import jax
import jax.numpy as jnp
from jax import lax
from jax.experimental import pallas as pl
from jax.experimental.pallas import tpu as pltpu
from jax.experimental.pallas import tpu_sc as plsc

BSZ = 4
N = 10000
PHY = 3
NP = 1024
K = 64
C = 32
KM = 16
L = 16

NCORE = 2
NSUB = 16
BPC = BSZ // NCORE
PPT = NP // NSUB
PAIRS = BPC * PPT


def _vsqrt(sv):
    iv = plsc.bitcast(sv, jnp.int32)
    iv = lax.shift_right_logical(iv, 1) + 0x1FBD1DF5
    y = plsc.bitcast(iv, jnp.float32)
    for _ in range(3):
        y = 0.5 * (y + sv / y)
    return y


def _sc_k1_body(gpl_hbm, cpl_hbm, eg_hbm, eq_hbm, xp_hbm, z_hbm,
                xhat_hbm, gauss_hbm,
                gpl_l, cpl_l, eg_l, eq_l, gauss_l,
                gidx0, gidx1, qg0, qg1, wg0, wg1, xr0, xr1,
                xhat_sh, sem0, sem1):
    c = lax.axis_index("c")
    s = lax.axis_index("s")
    iota = lax.iota(jnp.int32, L)
    gidx = (gidx0, gidx1)
    qg = (qg0, qg1)
    wg = (wg0, wg1)
    xr = (xr0, xr1)
    sems = (sem0, sem1)

    for d in range(4):
        for j in range(BPC):
            pltpu.sync_copy(gpl_hbm.at[pl.ds((d * BSZ + 2 * c + j) * N, N)],
                            gpl_l.at[pl.ds((d * BPC + j) * N, N)])
    pltpu.sync_copy(cpl_hbm, cpl_l)

    @pl.when(s == 0)
    def _():
        pltpu.sync_copy(z_hbm, xhat_sh)
    plsc.subcore_barrier()

    def phase_a(b_l, j, P):
        p = s * PPT + j
        b = 2 * c + b_l
        bwx = plsc.load_gather(cpl_l, [jnp.full((L,), 3 * NP, jnp.int32) + p])
        bwy = plsc.load_gather(cpl_l, [jnp.full((L,), 4 * NP, jnp.int32) + p])
        bwz = plsc.load_gather(cpl_l, [jnp.full((L,), 5 * NP, jnp.int32) + p])
        acc = jnp.zeros((L,), jnp.float32)
        for blk in range(K // L):
            ge = eg_l[pl.ds(j * K + blk * L, L)]
            qe = eq_l[pl.ds(j * K + blk * L, L)]
            geb = ge + b_l * N
            gx = plsc.load_gather(gpl_l, [geb])
            gy = plsc.load_gather(gpl_l, [geb + BPC * N])
            gz = plsc.load_gather(gpl_l, [geb + 2 * BPC * N])
            gw = plsc.load_gather(gpl_l, [geb + 3 * BPC * N])
            bx = plsc.load_gather(cpl_l, [qe])
            by = plsc.load_gather(cpl_l, [qe + NP])
            bz = plsc.load_gather(cpl_l, [qe + 2 * NP])
            dx = gx - bx
            dy = gy - by
            dz = gz - bz
            dw = bwx * dx * dx + bwy * dy * dy + bwz * dz * dz
            gv = jnp.exp(-dw)
            wv = gv * gw
            gauss_l[pl.ds(j * K + blk * L, L)] = gv
            acc = acc + wv * wv
            wg[P][pl.ds(blk * L, L)] = wv
            gidx[P][pl.ds(blk * L, L)] = ge + b * N
            qg[P][pl.ds(blk * L, L)] = qe + b_l * NP
        tot = jnp.sum(acc, axis=0)
        winv = 1.0 / (_vsqrt(jnp.full((L,), 0.0, jnp.float32) + tot) + 1e-5)
        for blk in range(K // L):
            wg[P][pl.ds(blk * L, L)] = wg[P][pl.ds(blk * L, L)] * winv

    def issue_g(P):
        return pltpu.async_copy(xp_hbm.at[gidx[P]], xr[P], sems[P])

    def wait_g(P):
        pltpu.make_async_copy(xp_hbm.at[gidx[P]], xr[P], sems[P]).wait()

    def scale_scatter(P):
        for blk in range(K // L):
            ridx = iota + blk * L
            wv = wg[P][pl.ds(blk * L, L)]
            for ch in range(C):
                cfull = jnp.full((L,), ch, jnp.int32)
                v = plsc.load_gather(xr[P], [ridx, cfull])
                plsc.store_scatter(xr[P], [ridx, cfull], v * wv)
        pltpu.sync_copy(xr[P], xhat_sh.at[qg[P]], add=True)

    for b_l in range(BPC):
        b = 2 * c + b_l
        base = (b * NP + s * PPT) * K
        pltpu.sync_copy(eg_hbm.at[pl.ds(base, PPT * K)], eg_l)
        pltpu.sync_copy(eq_hbm.at[pl.ds(base, PPT * K)], eq_l)

        phase_a(b_l, 0, 0)
        issue_g(0)
        phase_a(b_l, 1, 1)
        issue_g(1)

        def steady(t, _, b_l=b_l):
            wait_g(0)
            scale_scatter(0)
            phase_a(b_l, 2 * t + 2, 0)
            issue_g(0)
            wait_g(1)
            scale_scatter(1)
            phase_a(b_l, 2 * t + 3, 1)
            issue_g(1)
            return 0

        lax.fori_loop(0, PPT // 2 - 1, steady, 0)
        wait_g(0)
        scale_scatter(0)
        wait_g(1)
        scale_scatter(1)
        pltpu.sync_copy(gauss_l, gauss_hbm.at[pl.ds(base, PPT * K)])

    plsc.subcore_barrier()
    rows = BPC * NP // NSUB
    pltpu.sync_copy(xhat_sh.at[pl.ds(s * rows, rows)],
                    xhat_hbm.at[pl.ds(c * BPC * NP + s * rows, rows)])


def _sc_k3_body(y_hbm, gauss_hbm, eg_hbm, eq_hbm, z_hbm,
                out_hbm,
                y_l, eg_l, eq_l, gauss_l, gidx_l, rows_v,
                out_sh, sem):
    c = lax.axis_index("c")
    s = lax.axis_index("s")
    iota = lax.iota(jnp.int32, L)
    zrows = BPC * N // NSUB

    pltpu.sync_copy(y_hbm.at[pl.ds(c * BPC * NP * C, BPC * NP * C)], y_l)

    @pl.when(s == 0)
    def _():
        pltpu.sync_copy(z_hbm, out_sh)
    plsc.subcore_barrier()

    def pair_body(b_l, j):
        for blk in range(K // L):
            ge = eg_l[pl.ds(j * K + blk * L, L)]
            qrow = (eq_l[pl.ds(j * K + blk * L, L)] + b_l * NP) * C
            gv = gauss_l[pl.ds(j * K + blk * L, L)]
            ridx = iota + blk * L
            gidx_l[pl.ds(blk * L, L)] = ge + b_l * N
            for ch in range(C):
                cfull = jnp.full((L,), ch, jnp.int32)
                v = plsc.load_gather(y_l, [qrow + ch])
                plsc.store_scatter(rows_v, [ridx, cfull], v * gv)
        pltpu.sync_copy(rows_v, out_sh.at[gidx_l], add=True)
        return 0

    for b_l in range(BPC):
        b = 2 * c + b_l
        base = (b * NP + s * PPT) * K
        pltpu.sync_copy(eg_hbm.at[pl.ds(base, PPT * K)], eg_l)
        pltpu.sync_copy(eq_hbm.at[pl.ds(base, PPT * K)], eq_l)
        pltpu.sync_copy(gauss_hbm.at[pl.ds(base, PPT * K)], gauss_l)
        lax.fori_loop(0, PPT, lambda i, _, b_l=b_l: pair_body(b_l, i), 0)

    plsc.subcore_barrier()
    pltpu.sync_copy(out_sh.at[pl.ds(s * zrows, zrows)],
                    out_hbm.at[pl.ds(c * BPC * N + s * zrows, zrows)])


def _tc_k2_body(xh_ref, w2_ref, dt_ref, y_ref):
    t = jnp.dot(xh_ref[...], w2_ref[...], preferred_element_type=jnp.float32)
    acc = jnp.zeros_like(y_ref)
    for j in range(KM):
        acc = acc + t[:, j * C:(j + 1) * C] * dt_ref[:, j:j + 1]
    y_ref[...] = acc


@jax.jit
def kernel(x, grid, grid_weight, edge_grid, edge_Gauss, basepts, base_weight, D, weights):
    gpl = jnp.concatenate([jnp.transpose(grid, (2, 0, 1)),
                           grid_weight[None]], axis=0).reshape(-1)
    cpl = jnp.concatenate([basepts.T, base_weight.T,
                           jnp.zeros((2, NP), jnp.float32)], axis=0)
    cpl_flat = cpl.reshape(-1)
    eg_flat = edge_grid.reshape(-1)
    eq_flat = edge_Gauss.reshape(-1)
    xp = jnp.transpose(x, (0, 2, 1)).reshape(BSZ * N, C)
    z1 = jnp.zeros((BPC * NP, C), jnp.float32)
    z3 = jnp.zeros((BPC * N, C), jnp.float32)

    mesh = plsc.VectorSubcoreMesh(core_axis_name="c", subcore_axis_name="s")
    sc_params = pltpu.CompilerParams(needs_layout_passes=False,
                                     use_tc_tiling_on_sc=False)

    k1 = pl.kernel(
        _sc_k1_body,
        compiler_params=sc_params,
        out_type=[jax.ShapeDtypeStruct((BSZ * NP, C), jnp.float32),
                  jax.ShapeDtypeStruct((BSZ * NP * K,), jnp.float32)],
        mesh=mesh,
        scratch_types=[
            pltpu.VMEM((4 * BPC * N,), jnp.float32),
            pltpu.VMEM((8 * NP,), jnp.float32),
            pltpu.VMEM((PPT * K,), jnp.int32),
            pltpu.VMEM((PPT * K,), jnp.int32),
            pltpu.VMEM((PPT * K,), jnp.float32),
            pltpu.VMEM((K,), jnp.int32),
            pltpu.VMEM((K,), jnp.int32),
            pltpu.VMEM((K,), jnp.int32),
            pltpu.VMEM((K,), jnp.int32),
            pltpu.VMEM((K,), jnp.float32),
            pltpu.VMEM((K,), jnp.float32),
            pltpu.VMEM((K, C), jnp.float32),
            pltpu.VMEM((K, C), jnp.float32),
            pltpu.MemorySpace.VMEM_SHARED((BPC * NP, C), jnp.float32),
            pltpu.SemaphoreType.DMA,
            pltpu.SemaphoreType.DMA,
        ],
    )
    x_hat, gauss = k1(gpl, cpl_flat, eg_flat, eq_flat, xp, z1)

    w2 = jnp.transpose(weights, (0, 2, 1)).reshape(C, KM * C)
    dt = D.T
    RB = 512
    y = pl.pallas_call(
        _tc_k2_body,
        grid=(BSZ * NP // RB,),
        in_specs=[
            pl.BlockSpec((RB, C), lambda i: (i, 0)),
            pl.BlockSpec((C, KM * C), lambda i: (0, 0)),
            pl.BlockSpec((RB, KM), lambda i: (i % (NP // RB), 0)),
        ],
        out_specs=pl.BlockSpec((RB, C), lambda i: (i, 0)),
        out_shape=jax.ShapeDtypeStruct((BSZ * NP, C), jnp.float32),
    )(x_hat, w2, dt)

    k3 = pl.kernel(
        _sc_k3_body,
        compiler_params=sc_params,
        out_type=jax.ShapeDtypeStruct((BSZ * N, C), jnp.float32),
        mesh=mesh,
        scratch_types=[
            pltpu.VMEM((BPC * NP * C,), jnp.float32),
            pltpu.VMEM((PPT * K,), jnp.int32),
            pltpu.VMEM((PPT * K,), jnp.int32),
            pltpu.VMEM((PPT * K,), jnp.float32),
            pltpu.VMEM((K,), jnp.int32),
            pltpu.VMEM((K, C), jnp.float32),
            pltpu.MemorySpace.VMEM_SHARED((BPC * N, C), jnp.float32),
            pltpu.SemaphoreType.DMA,
        ],
    )
    out = k3(y.reshape(-1), gauss, eg_flat, eq_flat, z3)
    return jnp.transpose(out.reshape(BSZ, N, C), (0, 2, 1))
